# Initial kernel scaffold; baseline (speedup 1.0000x reference)
import functools

import jax
import jax.numpy as jnp
from jax.experimental import pallas as pl


def _expand_body(idx_ref, rep_ref, src_ref, w_ref, b_ref, out_ref, pooled_ref,
                 *, seq: int, dim: int):
    flat = seq * dim
    bt = idx_ref.shape[0]
    idx_bf = idx_ref[...].astype(jnp.bfloat16)
    rep = jnp.dot(idx_bf, rep_ref[...], preferred_element_type=jnp.float32)
    kpat = jax.lax.broadcasted_iota(jnp.int32, (bt, flat), 1)
    kpat = kpat - dim * (kpat // dim)
    lookup = rep.astype(jnp.int32) * dim + kpat
    src = jnp.broadcast_to(src_ref[...], (bt, flat))
    vals = jnp.take_along_axis(src, lookup, axis=1)
    out_ref[...] = vals
    ft = vals[:, :dim]
    pooled = jnp.dot(ft, w_ref[...], preferred_element_type=jnp.float32)
    pooled_ref[...] = jnp.tanh(pooled + b_ref[...])


def kernel(inputs, table, W, b):
    batch, seq = inputs.shape
    vocab, dim = table.shape
    flat = seq * dim
    block_b = 512

    j = jnp.arange(flat, dtype=jnp.int32)
    s = jnp.arange(seq, dtype=jnp.int32)
    rep_mat = (s[:, None] == (j[None, :] // dim)).astype(jnp.bfloat16)
    table_flat = jnp.pad(table.reshape(-1), (0, 128 - vocab * dim))
    src_row = jnp.tile(table_flat, flat // 128 + 1)[:flat][None, :]

    grid = (batch // block_b,)
    out_flat, pooled = pl.pallas_call(
        functools.partial(_expand_body, seq=seq, dim=dim),
        grid=grid,
        in_specs=[
            pl.BlockSpec((block_b, seq), lambda i: (i, 0)),
            pl.BlockSpec((seq, flat), lambda i: (0, 0)),
            pl.BlockSpec((1, flat), lambda i: (0, 0)),
            pl.BlockSpec((dim, dim), lambda i: (0, 0)),
            pl.BlockSpec((1, dim), lambda i: (0, 0)),
        ],
        out_specs=[
            pl.BlockSpec((block_b, flat), lambda i: (i, 0)),
            pl.BlockSpec((block_b, dim), lambda i: (i, 0)),
        ],
        out_shape=[
            jax.ShapeDtypeStruct((batch, flat), jnp.float32),
            jax.ShapeDtypeStruct((batch, dim), jnp.float32),
        ],
    )(inputs, rep_mat, src_row, W, b[None, :])
    return out_flat.reshape(batch, seq, dim), pooled


# speedup vs baseline: 22.3444x; 22.3444x over previous
import functools

import jax
import jax.numpy as jnp
from jax.experimental import pallas as pl

_LANES = 128


def _expand_body(idx_ref, rep_ref, src_ref, w_ref, b_ref, out_ref, pooled_ref,
                 *, seq: int, dim: int):
    flat = seq * dim
    flatp = rep_ref.shape[1]
    bt = idx_ref.shape[0]
    idx_bf = idx_ref[...].astype(jnp.bfloat16)
    rep = jnp.dot(idx_bf, rep_ref[...], preferred_element_type=jnp.float32)
    repi = rep.astype(jnp.int32)
    src = jnp.broadcast_to(src_ref[...], (bt, _LANES))
    ft = None
    for j0 in range(0, flatp, _LANES):
        kcol = jax.lax.broadcasted_iota(jnp.int32, (bt, _LANES), 1) + j0
        kcol = kcol - dim * (kcol // dim)
        lookup = repi[:, j0:j0 + _LANES] * dim + kcol
        vals = jnp.take_along_axis(src, lookup, axis=1)
        w = min(_LANES, flat - j0)
        out_ref[:, j0:j0 + w] = vals[:, :w]
        if j0 == 0:
            ft = vals[:, :dim]
    pooled = jnp.dot(ft, w_ref[...], preferred_element_type=jnp.float32)
    pooled_ref[...] = jnp.tanh(pooled + b_ref[...])


def kernel(inputs, table, W, b):
    batch, seq = inputs.shape
    vocab, dim = table.shape
    flat = seq * dim
    flatp = ((flat + _LANES - 1) // _LANES) * _LANES
    block_b = 512

    j = jnp.arange(flatp, dtype=jnp.int32)
    s = jnp.arange(seq, dtype=jnp.int32)
    rep_mat = (s[:, None] == (j[None, :] // dim)).astype(jnp.bfloat16)
    src_row = jnp.pad(table.reshape(-1), (0, _LANES - vocab * dim))[None, :]

    grid = (batch // block_b,)
    out_flat, pooled = pl.pallas_call(
        functools.partial(_expand_body, seq=seq, dim=dim),
        grid=grid,
        in_specs=[
            pl.BlockSpec((block_b, seq), lambda i: (i, 0)),
            pl.BlockSpec((seq, flatp), lambda i: (0, 0)),
            pl.BlockSpec((1, _LANES), lambda i: (0, 0)),
            pl.BlockSpec((dim, dim), lambda i: (0, 0)),
            pl.BlockSpec((1, dim), lambda i: (0, 0)),
        ],
        out_specs=[
            pl.BlockSpec((block_b, flat), lambda i: (i, 0)),
            pl.BlockSpec((block_b, dim), lambda i: (i, 0)),
        ],
        out_shape=[
            jax.ShapeDtypeStruct((batch, flat), jnp.float32),
            jax.ShapeDtypeStruct((batch, dim), jnp.float32),
        ],
    )(inputs, rep_mat, src_row, W, b[None, :])
    return out_flat.reshape(batch, seq, dim), pooled
